# baseline (speedup 1.0000x reference)
import functools

import jax
import jax.numpy as jnp
from jax import lax
from jax.experimental import pallas as pl
from jax.experimental.pallas import tpu as pltpu
from jax.experimental.pallas import tpu_sc as plsc

NUM_FEATURES = 26
CAT_SIZE = 1000
D_EMBED = 128
BATCH = 16384
TOTAL = BATCH * NUM_FEATURES

NC = 2
NS = 16
NW = NC * NS
N_SPLIT = 2
BSPLIT = BATCH // N_SPLIT
PER_W = BSPLIT * NUM_FEATURES // NW
SAMP_W = BSPLIT // NW
G_IDX = 104
CH_ROWS = 416
SAMP_C = CH_ROWS // NUM_FEATURES
N_CH = PER_W // CH_ROWS
N_GRP = PER_W // 16
ADJ_OUTER = 8
ADJ_INNER = N_GRP // ADJ_OUTER


def _sc_embedding_gather(x1d, weight):
    mesh = plsc.VectorSubcoreMesh(core_axis_name="c", subcore_axis_name="s")

    @functools.partial(
        pl.kernel,
        mesh=mesh,
        out_type=jax.ShapeDtypeStruct((BSPLIT, NUM_FEATURES, D_EMBED), jnp.float32),
        scratch_types=[
            pltpu.VMEM((PER_W,), jnp.int32),
            pltpu.VMEM((CH_ROWS, D_EMBED), jnp.float32),
            pltpu.VMEM((CH_ROWS, D_EMBED), jnp.float32),
            pltpu.SemaphoreType.DMA,
            pltpu.SemaphoreType.DMA,
            pltpu.SemaphoreType.DMA,
            pltpu.SemaphoreType.DMA,
        ],
    )
    def body(x_hbm, w_hbm, out_hbm, idx_all, bufa, bufb, semga, semgb, semwa, semwb):
        wid = lax.axis_index("s") * NC + lax.axis_index("c")
        base = pl.multiple_of(wid * PER_W, PER_W)
        sbase = wid * SAMP_W

        pltpu.sync_copy(x_hbm.at[pl.ds(base, PER_W)], idx_all)

        lane = lax.iota(jnp.int32, 16)
        offs = [
            lax.rem(lane + (16 * k) % NUM_FEATURES, NUM_FEATURES) * CAT_SIZE
            for k in range(13)
        ]

        def adj_body(c, carry):
            g0 = c * ADJ_INNER
            for k in range(ADJ_INNER):
                sl = pl.ds(pl.multiple_of((g0 + k) * 16, 16), 16)
                idx_all[sl] = idx_all[sl] + offs[k % 13]
            return carry

        lax.fori_loop(0, ADJ_OUTER, adj_body, 0)

        def run_chunk(n, buf, semg, semw, first):
            f0 = pl.multiple_of(n * CH_ROWS, CH_ROWS)
            sb = sbase + n * SAMP_C
            wb = lambda: pltpu.async_copy(
                buf.reshape(SAMP_C, NUM_FEATURES, D_EMBED),
                out_hbm.at[pl.ds(sb, SAMP_C)],
                semw,
            )
            drain = lambda: pltpu.make_async_copy(
                buf.reshape(SAMP_C, NUM_FEATURES, D_EMBED),
                out_hbm.at[pl.ds(sb, SAMP_C)],
                semw,
            ).wait()
            if first is None:
                drain()
            else:
                pl.when(first)(drain)
            gathers = [
                pltpu.async_copy(
                    w_hbm.at[idx_all.at[pl.ds(f0 + q * G_IDX, G_IDX)]],
                    buf.at[pl.ds(q * G_IDX, G_IDX)],
                    semg,
                )
                for q in range(4)
            ]
            for g in gathers:
                g.wait()
            wb()

        def pair_body(c, carry):
            run_chunk(2 * c, bufa, semga, semwa, c > 0)
            run_chunk(2 * c + 1, bufb, semgb, semwb, c > 0)
            return carry

        lax.fori_loop(0, N_CH // 2, pair_body, 0)

        for buf, semw, n in ((bufa, semwa, N_CH - 2), (bufb, semwb, N_CH - 1)):
            sb = sbase + n * SAMP_C
            pltpu.make_async_copy(
                buf.reshape(SAMP_C, NUM_FEATURES, D_EMBED),
                out_hbm.at[pl.ds(sb, SAMP_C)],
                semw,
            ).wait()

    return body(x1d, weight)


def kernel(x_cat, weight):
    x1d = x_cat.reshape(TOTAL)
    pieces = [
        _sc_embedding_gather(
            lax.slice(x1d, (s * BSPLIT * NUM_FEATURES,),
                      ((s + 1) * BSPLIT * NUM_FEATURES,)),
            weight,
        )
        for s in range(N_SPLIT)
    ]
    return jnp.concatenate(pieces, axis=0) if N_SPLIT > 1 else pieces[0]


# speedup vs baseline: 1.5328x; 1.5328x over previous
import functools

import jax
import jax.numpy as jnp
from jax import lax
from jax.experimental import pallas as pl
from jax.experimental.pallas import tpu as pltpu
from jax.experimental.pallas import tpu_sc as plsc

NUM_FEATURES = 26
CAT_SIZE = 1000
D_EMBED = 128
BATCH = 16384
TOTAL = BATCH * NUM_FEATURES

NC = 2
NS = 16
NW = NC * NS
PER_W = TOTAL // NW
SAMP_W = BATCH // NW
G_IDX = 104
N_GATH = 2
CH_ROWS = N_GATH * G_IDX
SAMP_C = CH_ROWS // NUM_FEATURES
N_CH = PER_W // CH_ROWS
N_BUF = 4
N_GRP = PER_W // 16
ADJ_OUTER = 8
ADJ_INNER = N_GRP // ADJ_OUTER


def _sc_embedding_gather(x1d, weight):
    mesh = plsc.VectorSubcoreMesh(core_axis_name="c", subcore_axis_name="s")

    @functools.partial(
        pl.kernel,
        mesh=mesh,
        out_type=jax.ShapeDtypeStruct((BATCH, NUM_FEATURES, D_EMBED), jnp.float32),
        scratch_types=[pltpu.VMEM((PER_W,), jnp.int32)]
        + [pltpu.VMEM((CH_ROWS, D_EMBED), jnp.float32) for _ in range(N_BUF)]
        + [pltpu.SemaphoreType.DMA for _ in range(2 * N_BUF)],
    )
    def body(x_hbm, w_hbm, out_hbm, idx_all, *rest):
        bufs = rest[:N_BUF]
        semgs = rest[N_BUF:2 * N_BUF]
        semws = rest[2 * N_BUF:]
        wid = lax.axis_index("s") * NC + lax.axis_index("c")
        base = pl.multiple_of(wid * PER_W, PER_W)
        sbase = wid * SAMP_W

        pltpu.sync_copy(x_hbm.at[pl.ds(base, PER_W)], idx_all)

        lane = lax.iota(jnp.int32, 16)
        offs = [
            lax.rem(lane + (16 * k) % NUM_FEATURES, NUM_FEATURES) * CAT_SIZE
            for k in range(13)
        ]

        def adj_body(c, carry):
            g0 = c * ADJ_INNER
            for k in range(ADJ_INNER):
                sl = pl.ds(pl.multiple_of((g0 + k) * 16, 16), 16)
                idx_all[sl] = idx_all[sl] + offs[k % 13]
            return carry

        lax.fori_loop(0, ADJ_OUTER, adj_body, 0)

        def fire_gathers(n, j):
            f0 = pl.multiple_of(n * CH_ROWS, CH_ROWS)
            for q in range(N_GATH):
                pltpu.async_copy(
                    w_hbm.at[idx_all.at[pl.ds(f0 + q * G_IDX, G_IDX)]],
                    bufs[j].at[pl.ds(q * G_IDX, G_IDX)],
                    semgs[j],
                )

        def wait_gathers(n, j):
            f0 = pl.multiple_of(n * CH_ROWS, CH_ROWS)
            for q in range(N_GATH):
                pltpu.make_async_copy(
                    w_hbm.at[idx_all.at[pl.ds(f0 + q * G_IDX, G_IDX)]],
                    bufs[j].at[pl.ds(q * G_IDX, G_IDX)],
                    semgs[j],
                ).wait()

        def wb_copy(n, j):
            sb = sbase + n * SAMP_C
            return (
                bufs[j].reshape(SAMP_C, NUM_FEATURES, D_EMBED),
                out_hbm.at[pl.ds(sb, SAMP_C)],
                semws[j],
            )

        def ring_body(c, carry):
            for j in range(N_BUF):
                n = N_BUF * c + j
                drain = lambda: pltpu.make_async_copy(*wb_copy(n - N_BUF, j)).wait()
                pl.when(c > 0)(drain)
                fire_gathers(n, j)
                jp = (j - 1) % N_BUF
                def finish(n=n, jp=jp):
                    wait_gathers(n - 1, jp)
                    pltpu.async_copy(*wb_copy(n - 1, jp))
                if j == 0:
                    pl.when(c > 0)(finish)
                else:
                    finish()
            return carry

        lax.fori_loop(0, N_CH // N_BUF, ring_body, 0)

        wait_gathers(N_CH - 1, (N_CH - 1) % N_BUF)
        pltpu.async_copy(*wb_copy(N_CH - 1, (N_CH - 1) % N_BUF))
        for j in range(N_BUF):
            n = N_CH - N_BUF + j
            pltpu.make_async_copy(*wb_copy(n, j)).wait()

    return body(x1d, weight)


def kernel(x_cat, weight):
    return _sc_embedding_gather(x_cat.reshape(TOTAL), weight)


# speedup vs baseline: 1.5393x; 1.0043x over previous
import functools

import jax
import jax.numpy as jnp
from jax import lax
from jax.experimental import pallas as pl
from jax.experimental.pallas import tpu as pltpu
from jax.experimental.pallas import tpu_sc as plsc

NUM_FEATURES = 26
CAT_SIZE = 1000
D_EMBED = 128
BATCH = 16384
TOTAL = BATCH * NUM_FEATURES

NC = 2
NS = 16
NW = NC * NS
PER_W = TOTAL // NW
SAMP_W = BATCH // NW
G_IDX = 104
CH_ROWS = G_IDX
SAMP_C = CH_ROWS // NUM_FEATURES
N_CH = PER_W // CH_ROWS
N_BUF = 8
DEPTH = 4
N_GRP = PER_W // 16
ADJ_OUTER = 8
ADJ_INNER = N_GRP // ADJ_OUTER


def _sc_embedding_gather(x1d, weight):
    mesh = plsc.VectorSubcoreMesh(core_axis_name="c", subcore_axis_name="s")

    @functools.partial(
        pl.kernel,
        mesh=mesh,
        out_type=jax.ShapeDtypeStruct((BATCH, NUM_FEATURES, D_EMBED), jnp.float32),
        scratch_types=[pltpu.VMEM((PER_W,), jnp.int32)]
        + [pltpu.VMEM((CH_ROWS, D_EMBED), jnp.float32) for _ in range(N_BUF)]
        + [pltpu.SemaphoreType.DMA for _ in range(2 * N_BUF)],
    )
    def body(x_hbm, w_hbm, out_hbm, idx_all, *rest):
        bufs = rest[:N_BUF]
        semgs = rest[N_BUF:2 * N_BUF]
        semws = rest[2 * N_BUF:]
        wid = lax.axis_index("s") * NC + lax.axis_index("c")
        base = pl.multiple_of(wid * PER_W, PER_W)
        sbase = wid * SAMP_W

        pltpu.sync_copy(x_hbm.at[pl.ds(base, PER_W)], idx_all)

        lane = lax.iota(jnp.int32, 16)
        offs = [
            lax.rem(lane + (16 * k) % NUM_FEATURES, NUM_FEATURES) * CAT_SIZE
            for k in range(13)
        ]

        def adj_body(c, carry):
            g0 = c * ADJ_INNER
            for k in range(ADJ_INNER):
                sl = pl.ds(pl.multiple_of((g0 + k) * 16, 16), 16)
                idx_all[sl] = idx_all[sl] + offs[k % 13]
            return carry

        lax.fori_loop(0, ADJ_OUTER, adj_body, 0)

        def gather_copy(n, b):
            f0 = pl.multiple_of(n * CH_ROWS, CH_ROWS)
            return (
                w_hbm.at[idx_all.at[pl.ds(f0, G_IDX)]],
                bufs[b],
                semgs[b],
            )

        def wb_copy(n, b):
            sb = sbase + n * SAMP_C
            return (
                bufs[b].reshape(SAMP_C, NUM_FEATURES, D_EMBED),
                out_hbm.at[pl.ds(sb, SAMP_C)],
                semws[b],
            )

        for n in range(DEPTH):
            pltpu.async_copy(*gather_copy(n, n))

        def ring_body(c, carry):
            for j in range(N_BUF):
                n = N_BUF * c + j
                pltpu.make_async_copy(*gather_copy(n, j)).wait()
                pltpu.async_copy(*wb_copy(n, j))
                bn = (j + DEPTH) % N_BUF
                drain = lambda: pltpu.make_async_copy(
                    *wb_copy(n - DEPTH, bn)
                ).wait()
                def fire(n=n, bn=bn):
                    pltpu.async_copy(*gather_copy(n + DEPTH, bn))
                if j < DEPTH:
                    pl.when(c > 0)(drain)
                    fire()
                else:
                    drain()
                    pl.when(c < N_CH // N_BUF - 1)(fire)
            return carry

        lax.fori_loop(0, N_CH // N_BUF, ring_body, 0)

        for n in range(N_CH - DEPTH, N_CH):
            pltpu.make_async_copy(*wb_copy(n, n % N_BUF)).wait()

    return body(x1d, weight)


def kernel(x_cat, weight):
    return _sc_embedding_gather(x_cat.reshape(TOTAL), weight)
